# Initial kernel scaffold; baseline (speedup 1.0000x reference)
import jax
import jax.numpy as jnp
from jax.experimental import pallas as pl


def kernel(x, table):
    raise NotImplementedError("write your pallas kernel here")


# speedup vs baseline: 2.2834x; 2.2834x over previous
import functools

import jax
import jax.numpy as jnp
import numpy as np
from jax import lax
from jax.experimental import pallas as pl
from jax.experimental.pallas import tpu as pltpu
from jax.experimental.pallas import tpu_sc as plsc

D_MODEL = 128
VOCAB = 1000
SCALE = np.sqrt(np.float32(D_MODEL)).astype(np.float32)

NUM_CORES = 2
NUM_SUBCORES = 16
NUM_WORKERS = NUM_CORES * NUM_SUBCORES
CHUNK = 128
LANES = 16


def _embed_kernel(n_rows: int):
    rows_per_worker = n_rows // NUM_WORKERS
    n_chunks = rows_per_worker // CHUNK
    mesh = plsc.VectorSubcoreMesh(core_axis_name="c", subcore_axis_name="s")

    @functools.partial(
        pl.kernel,
        out_type=jax.ShapeDtypeStruct((n_rows, D_MODEL), jnp.float32),
        mesh=mesh,
        scratch_types=[
            pltpu.VMEM((CHUNK,), jnp.int32),
            pltpu.VMEM((CHUNK, D_MODEL), jnp.float32),
            pltpu.SemaphoreType.DMA,
        ],
    )
    def k(x_hbm, table_hbm, out_hbm, idx_v, rows_v, sem):
        wid = lax.axis_index("s") * NUM_CORES + lax.axis_index("c")
        wbase = wid * rows_per_worker

        def chunk_body(j, carry):
            base = wbase + j * CHUNK
            pltpu.sync_copy(x_hbm.at[pl.ds(base, CHUNK)], idx_v)
            pltpu.async_copy(table_hbm.at[idx_v], rows_v, sem).wait()

            def scale_body(i, c):
                for jj in range(D_MODEL // LANES):
                    sl = pl.ds(jj * LANES, LANES)
                    rows_v[i, sl] = rows_v[i, sl] * SCALE
                return c

            lax.fori_loop(0, CHUNK, scale_body, 0)
            pltpu.sync_copy(rows_v, out_hbm.at[pl.ds(base, CHUNK)])
            return carry

        lax.fori_loop(0, n_chunks, chunk_body, 0)

    return k


@jax.jit
def kernel(x, table):
    b, h = x.shape
    flat = x.reshape(b * h)
    out = _embed_kernel(b * h)(flat, table)
    return out.reshape(b, h, D_MODEL)


# speedup vs baseline: 3.5051x; 1.5350x over previous
import functools

import jax
import jax.numpy as jnp
import numpy as np
from jax import lax
from jax.experimental import pallas as pl
from jax.experimental.pallas import tpu as pltpu
from jax.experimental.pallas import tpu_sc as plsc

D_MODEL = 128
VOCAB = 1000
SCALE = np.sqrt(np.float32(D_MODEL)).astype(np.float32)

NUM_CORES = 2
NUM_SUBCORES = 16
NUM_WORKERS = NUM_CORES * NUM_SUBCORES
CHUNK = 128
LANES = 16
ROWS_A = 64


def _embed_kernel(n_rows: int):
    rows_per_worker = n_rows // NUM_WORKERS
    n_chunks = rows_per_worker // CHUNK
    mesh = plsc.VectorSubcoreMesh(core_axis_name="c", subcore_axis_name="s")

    @functools.partial(
        pl.kernel,
        out_type=jax.ShapeDtypeStruct((n_rows, D_MODEL), jnp.float32),
        mesh=mesh,
        scratch_types=[
            pltpu.VMEM_SHARED((VOCAB, D_MODEL), jnp.float32),
            pltpu.VMEM((n_chunks, CHUNK), jnp.int32),
            pltpu.VMEM((CHUNK, D_MODEL), jnp.float32),
            pltpu.VMEM((CHUNK, D_MODEL), jnp.float32),
            pltpu.SemaphoreType.DMA,
            pltpu.SemaphoreType.DMA,
            pltpu.SemaphoreType.DMA,
            pltpu.SemaphoreType.DMA,
            pltpu.SemaphoreType.DMA,
        ],
    )
    def k(x_hbm, table_hbm, out_hbm, tbl_sh, idx_v, rows0, rows1,
          gsem0, gsem1, ssem0, ssem1, isem):
        cid = lax.axis_index("c")
        sid = lax.axis_index("s")
        wid = sid * NUM_CORES + cid
        wbase = wid * rows_per_worker
        rows = (rows0, rows1)
        gsem = (gsem0, gsem1)
        ssem = (ssem0, ssem1)

        idx_copy = pltpu.async_copy(
            x_hbm.at[pl.ds(wid * n_chunks, n_chunks)], idx_v, isem)

        abase = jnp.minimum(sid * ROWS_A, VOCAB - ROWS_A)
        stage = rows0.at[pl.ds(0, ROWS_A)]
        pltpu.sync_copy(table_hbm.at[pl.ds(abase, ROWS_A)], stage)

        def scale_body(i, c):
            for jj in range(D_MODEL // LANES):
                sl = pl.ds(jj * LANES, LANES)
                rows0[i, sl] = rows0[i, sl] * SCALE
            return c

        lax.fori_loop(0, ROWS_A, scale_body, 0)
        pltpu.sync_copy(stage, tbl_sh.at[pl.ds(abase, ROWS_A)])
        idx_copy.wait()
        plsc.subcore_barrier()

        def fire_gather(j, b):
            pltpu.async_copy(tbl_sh.at[idx_v.at[j]], rows[b], gsem[b])

        def wait_gather(b):
            pltpu.make_async_copy(
                tbl_sh.at[idx_v.at[0]], rows[b], gsem[b]).wait()

        def fire_store(j, b):
            pltpu.async_copy(
                rows[b], out_hbm.at[pl.ds(wbase + j * CHUNK, CHUNK)], ssem[b])

        def wait_store(b):
            pltpu.make_async_copy(
                rows[b], out_hbm.at[pl.ds(wbase, CHUNK)], ssem[b]).wait()

        fire_gather(0, 0)
        fire_gather(1, 1)
        wait_gather(0)
        fire_store(0, 0)
        wait_store(0)
        fire_gather(2, 0)
        wait_gather(1)
        fire_store(1, 1)

        def pair_body(j2, c):
            j = 2 * j2
            wait_store(1)
            fire_gather(j + 1, 1)
            wait_gather(0)
            fire_store(j, 0)
            wait_store(0)
            fire_gather(j + 2, 0)
            wait_gather(1)
            fire_store(j + 1, 1)
            return c

        lax.fori_loop(1, n_chunks // 2 - 1, pair_body, 0)

        wait_store(1)
        fire_gather(n_chunks - 1, 1)
        wait_gather(0)
        fire_store(n_chunks - 2, 0)
        wait_gather(1)
        fire_store(n_chunks - 1, 1)
        wait_store(0)
        wait_store(1)

    return k


@jax.jit
def kernel(x, table):
    b, h = x.shape
    n_rows = b * h
    flat = x.reshape(n_rows // CHUNK, CHUNK)
    out = _embed_kernel(n_rows)(flat, table)
    return out.reshape(b, h, D_MODEL)


# speedup vs baseline: 6.7842x; 1.9355x over previous
import functools

import jax
import jax.numpy as jnp
import numpy as np
from jax import lax
from jax.experimental import pallas as pl
from jax.experimental.pallas import tpu as pltpu
from jax.experimental.pallas import tpu_sc as plsc

D_MODEL = 128
VOCAB = 1000
SCALE = np.sqrt(np.float32(D_MODEL)).astype(np.float32)

NUM_CORES = 2
NUM_SUBCORES = 16
NUM_WORKERS = NUM_CORES * NUM_SUBCORES
LANES = 16
ROWS_A = 64
HPAD = 56
CB = 4


def _embed_kernel(batch: int, hist: int):
    b_per_w = batch // NUM_WORKERS
    n_chunks = b_per_w // CB
    idx_per_w = b_per_w * HPAD
    mesh = plsc.VectorSubcoreMesh(core_axis_name="c", subcore_axis_name="s")

    @functools.partial(
        pl.kernel,
        out_type=jax.ShapeDtypeStruct((batch, hist, D_MODEL), jnp.float32),
        mesh=mesh,
        compiler_params=pltpu.CompilerParams(use_tc_tiling_on_sc=True),
        scratch_types=[
            pltpu.VMEM_SHARED((VOCAB, D_MODEL), jnp.float32),
            pltpu.VMEM((idx_per_w,), jnp.int32),
            pltpu.VMEM((CB * HPAD, D_MODEL), jnp.float32),
            pltpu.VMEM((CB * HPAD, D_MODEL), jnp.float32),
            pltpu.SemaphoreType.DMA,
            pltpu.SemaphoreType.DMA,
            pltpu.SemaphoreType.DMA,
            pltpu.SemaphoreType.DMA,
            pltpu.SemaphoreType.DMA,
        ],
    )
    def k(x_hbm, table_hbm, out_hbm, tbl_sh, idx_v, rows0, rows1,
          gsem0, gsem1, ssem0, ssem1, isem):
        cid = lax.axis_index("c")
        sid = lax.axis_index("s")
        wid = sid * NUM_CORES + cid
        bbase = wid * b_per_w
        rows = (rows0, rows1)
        gsem = (gsem0, gsem1)
        ssem = (ssem0, ssem1)

        idx_copy = pltpu.async_copy(
            x_hbm.at[pl.ds(wid * idx_per_w, idx_per_w)], idx_v, isem)

        abase = jnp.minimum(sid * ROWS_A, VOCAB - ROWS_A)
        stage = rows0.at[pl.ds(0, ROWS_A)]
        pltpu.sync_copy(table_hbm.at[pl.ds(abase, ROWS_A)], stage)

        def scale_body(i, c):
            for jj in range(D_MODEL // LANES):
                sl = pl.ds(jj * LANES, LANES)
                rows0[i, sl] = rows0[i, sl] * SCALE
            return c

        lax.fori_loop(0, ROWS_A, scale_body, 0)
        pltpu.sync_copy(stage, tbl_sh.at[pl.ds(abase, ROWS_A)])
        idx_copy.wait()
        plsc.subcore_barrier()

        def fire_gather(j, b):
            for i in range(CB):
                pltpu.async_copy(
                    tbl_sh.at[idx_v.at[pl.ds((j * CB + i) * HPAD, HPAD)]],
                    rows[b].at[pl.ds(i * HPAD, HPAD)], gsem[b])

        def wait_gather(b):
            for _ in range(CB):
                pltpu.make_async_copy(
                    tbl_sh.at[idx_v.at[pl.ds(0, HPAD)]],
                    rows[b].at[pl.ds(0, HPAD)], gsem[b]).wait()

        def fire_store(j, b):
            for i in range(CB):
                pltpu.async_copy(
                    rows[b].at[pl.ds(i * HPAD, hist)],
                    out_hbm.at[bbase + j * CB + i], ssem[b])

        def wait_store(b):
            for _ in range(CB):
                pltpu.make_async_copy(
                    rows[b].at[pl.ds(0, hist)], out_hbm.at[0], ssem[b]).wait()

        fire_gather(0, 0)
        fire_gather(1, 1)
        wait_gather(0)
        fire_store(0, 0)
        wait_store(0)
        fire_gather(2, 0)
        wait_gather(1)
        fire_store(1, 1)

        def pair_body(j2, c):
            j = 2 * j2
            wait_store(1)
            fire_gather(j + 1, 1)
            wait_gather(0)
            fire_store(j, 0)
            wait_store(0)
            fire_gather(j + 2, 0)
            wait_gather(1)
            fire_store(j + 1, 1)
            return c

        lax.fori_loop(1, n_chunks // 2 - 1, pair_body, 0)

        wait_store(1)
        fire_gather(n_chunks - 1, 1)
        wait_gather(0)
        fire_store(n_chunks - 2, 0)
        wait_gather(1)
        fire_store(n_chunks - 1, 1)
        wait_store(0)
        wait_store(1)

    return k


@jax.jit
def kernel(x, table):
    b, h = x.shape
    xp = jnp.pad(x, ((0, 0), (0, HPAD - h))).reshape(b * HPAD)
    return _embed_kernel(b, h)(xp, table)


# speedup vs baseline: 18.7931x; 2.7701x over previous
import functools

import jax
import jax.numpy as jnp
import numpy as np
from jax import lax
from jax.experimental import pallas as pl
from jax.experimental.pallas import tpu as pltpu
from jax.experimental.pallas import tpu_sc as plsc

D_MODEL = 128
VOCAB = 1000
SCALE = np.sqrt(np.float32(D_MODEL)).astype(np.float32)

NUM_CORES = 2
NUM_SUBCORES = 16
NUM_WORKERS = NUM_CORES * NUM_SUBCORES
CHUNK = 128
LANES = 16
ROWS_A = 64
NBUF = 4


def _embed_kernel(n_rows: int):
    rows_per_worker = n_rows // NUM_WORKERS
    n_chunks = rows_per_worker // CHUNK
    mesh = plsc.VectorSubcoreMesh(core_axis_name="c", subcore_axis_name="s")

    @functools.partial(
        pl.kernel,
        out_type=jax.ShapeDtypeStruct((n_rows, D_MODEL), jnp.float32),
        mesh=mesh,
        compiler_params=pltpu.CompilerParams(use_tc_tiling_on_sc=True),
        scratch_types=[
            pltpu.VMEM_SHARED((VOCAB, D_MODEL), jnp.float32),
            pltpu.VMEM((n_chunks, CHUNK), jnp.int32),
        ]
        + [pltpu.VMEM((CHUNK, D_MODEL), jnp.float32)] * NBUF
        + [pltpu.SemaphoreType.DMA] * (2 * NBUF + 1),
    )
    def k(x_hbm, table_hbm, out_hbm, tbl_sh, idx_v, r0, r1, r2, r3,
          g0, g1, g2, g3, s0, s1, s2, s3, isem):
        cid = lax.axis_index("c")
        sid = lax.axis_index("s")
        wid = sid * NUM_CORES + cid
        wbase = wid * rows_per_worker
        rows = (r0, r1, r2, r3)
        gsem = (g0, g1, g2, g3)
        ssem = (s0, s1, s2, s3)

        idx_copy = pltpu.async_copy(
            x_hbm.at[pl.ds(wid * n_chunks, n_chunks)], idx_v, isem)

        abase = jnp.minimum(sid * ROWS_A, VOCAB - ROWS_A)
        stage = r0.at[pl.ds(0, ROWS_A)]
        pltpu.sync_copy(table_hbm.at[pl.ds(abase, ROWS_A)], stage)

        def scale_body(i, c):
            for jj in range(D_MODEL // LANES):
                sl = pl.ds(jj * LANES, LANES)
                r0[i, sl] = r0[i, sl] * SCALE
            return c

        lax.fori_loop(0, ROWS_A, scale_body, 0)
        pltpu.sync_copy(stage, tbl_sh.at[pl.ds(abase, ROWS_A)])
        idx_copy.wait()
        plsc.subcore_barrier()

        def fire_gather(j, b):
            pltpu.async_copy(tbl_sh.at[idx_v.at[j]], rows[b], gsem[b])

        def wait_gather(b):
            pltpu.make_async_copy(
                tbl_sh.at[idx_v.at[0]], rows[b], gsem[b]).wait()

        def fire_store(j, b):
            pltpu.async_copy(
                rows[b], out_hbm.at[pl.ds(wbase + j * CHUNK, CHUNK)], ssem[b])

        def wait_store(b):
            pltpu.make_async_copy(
                rows[b], out_hbm.at[pl.ds(wbase, CHUNK)], ssem[b]).wait()

        def step(j, b, fire_ahead=True, drain=True):
            nb = (b + 2) % NBUF
            if fire_ahead:
                if drain:
                    wait_store(nb)
                fire_gather(j + 2, nb)
            wait_gather(b)
            fire_store(j, b)

        fire_gather(0, 0)
        fire_gather(1, 1)
        step(0, 0, drain=False)
        step(1, 1, drain=False)
        step(2, 2)
        step(3, 3)

        def quad_body(j2, c):
            j = 4 * j2
            step(j + 0, 0)
            step(j + 1, 1)
            step(j + 2, 2)
            step(j + 3, 3)
            return c

        lax.fori_loop(1, n_chunks // 4 - 1, quad_body, 0)

        base = n_chunks - 4
        step(base + 0, 0)
        step(base + 1, 1)
        step(base + 2, 2, fire_ahead=False)
        step(base + 3, 3, fire_ahead=False)
        for b in range(NBUF):
            wait_store(b)

    return k


@jax.jit
def kernel(x, table):
    b, h = x.shape
    n_rows = b * h
    flat = x.T.reshape(n_rows // CHUNK, CHUNK)
    out = _embed_kernel(n_rows)(flat, table)
    return out.reshape(h, b, D_MODEL).transpose(1, 0, 2)
